# Initial kernel scaffold; baseline (speedup 1.0000x reference)
import jax
import jax.numpy as jnp
from jax.experimental import pallas as pl


def kernel(x, edge_index, W11, b11, W12, b12, W21, b21, W22, b22, W31, b31, W32, b32):
    raise NotImplementedError("write your pallas kernel here")


# speedup vs baseline: 1.3106x; 1.3106x over previous
import functools
import jax
import jax.numpy as jnp
from jax import lax
from jax.experimental import pallas as pl
from jax.experimental.pallas import tpu as pltpu, tpu_sc as plsc

N = 10000
D = 256
DH = D // 2
E = 160000

NC = 2
NS = 16
HALF = N // NC
TPT = 320
ACC_ROWS = NS * TPT
DUMMY = HALF

BATCH = 128
E_PER_TILE = 10240
E_PAD = NS * E_PER_TILE
NB = E_PER_TILE // BATCH


def _sc_aggregate(r_lo, r_hi, src, dst, zeros):
    mesh = plsc.VectorSubcoreMesh(core_axis_name="c", subcore_axis_name="s")

    @functools.partial(
        pl.kernel,
        out_type=[jax.ShapeDtypeStruct((NC * ACC_ROWS, DH), jnp.float32)] * 2,
        mesh=mesh,
        scratch_types=[
            pltpu.VMEM((E_PER_TILE,), jnp.int32),
            pltpu.VMEM((E_PER_TILE,), jnp.int32),
            pltpu.VMEM((BATCH,), jnp.int32),
            pltpu.VMEM((BATCH, DH), jnp.float32),
            pltpu.VMEM_SHARED((ACC_ROWS, DH), jnp.float32),
            pltpu.SemaphoreType.DMA,
        ],
    )
    def k(rlo_hbm, rhi_hbm, src_hbm, dst_hbm, zeros_hbm, outlo_hbm, outhi_hbm,
          src_v, dst_v, idx_v, rows_v, acc, sem):
        c = lax.axis_index("c")
        s = lax.axis_index("s")
        lo = c * HALF

        base = s * E_PER_TILE
        pltpu.sync_copy(src_hbm.at[pl.ds(base, E_PER_TILE)], src_v)
        pltpu.sync_copy(dst_hbm.at[pl.ds(base, E_PER_TILE)], dst_v)

        for r_hbm, out_hbm in ((rlo_hbm, outlo_hbm), (rhi_hbm, outhi_hbm)):
            pltpu.sync_copy(zeros_hbm, acc.at[pl.ds(s * TPT, TPT)])
            plsc.subcore_barrier()

            def body(b, carry):
                eb = b * BATCH
                for g in range(BATCH // 16):
                    d = dst_v[pl.ds(eb + g * 16, 16)]
                    inr = (d >= lo) & (d < lo + HALF)
                    idx_v[pl.ds(g * 16, 16)] = jnp.where(inr, d - lo, DUMMY)
                pltpu.async_copy(r_hbm.at[src_v.at[pl.ds(eb, BATCH)]],
                                 rows_v, sem).wait()
                pltpu.sync_copy(rows_v, acc.at[idx_v], add=True)
                return carry

            lax.fori_loop(0, NB, body, None)
            plsc.subcore_barrier()

            pltpu.sync_copy(acc.at[pl.ds(s * TPT, TPT)],
                            out_hbm.at[pl.ds(c * ACC_ROWS + s * TPT, TPT)])
            plsc.subcore_barrier()

    return k(r_lo, r_hi, src, dst, zeros)


def _relu_body(x_ref, lo_ref, hi_ref):
    r = jnp.maximum(x_ref[...], 0.0)
    lo_ref[...] = r[:, :DH]
    hi_ref[...] = r[:, DH:]


def _relu_tc(x):
    blk = N // 10
    return pl.pallas_call(
        _relu_body,
        out_shape=[jax.ShapeDtypeStruct((N, DH), jnp.float32)] * 2,
        grid=(10,),
        in_specs=[pl.BlockSpec((blk, D), lambda i: (i, 0))],
        out_specs=[pl.BlockSpec((blk, DH), lambda i: (i, 0))] * 2,
    )(x)


def _mlp_tc(h, aggr, w1, b1, w2, b2, do_elu, want_relu):
    blk = N // 10
    n_out = 3 if want_relu else 1
    out_shape = [jax.ShapeDtypeStruct((N, D), jnp.float32)]
    out_specs = [pl.BlockSpec((blk, D), lambda i: (i, 0))]
    if want_relu:
        out_shape += [jax.ShapeDtypeStruct((N, DH), jnp.float32)] * 2
        out_specs += [pl.BlockSpec((blk, DH), lambda i: (i, 0))] * 2

    def body(h_ref, a_ref, w1_ref, b1_ref, w2_ref, b2_ref, o_ref, *r_refs):
        y = h_ref[...] + a_ref[...]
        t = jnp.dot(y, w1_ref[...], preferred_element_type=jnp.float32)
        t = jnp.maximum(t + b1_ref[...], 0.0)
        z = jnp.dot(t, w2_ref[...], preferred_element_type=jnp.float32)
        z = z + b2_ref[...]
        if do_elu:
            z = jnp.where(z > 0.0, z, jnp.exp(jnp.minimum(z, 0.0)) - 1.0)
        o_ref[...] = z
        if r_refs:
            r = jnp.maximum(z, 0.0)
            r_refs[0][...] = r[:, :DH]
            r_refs[1][...] = r[:, DH:]

    res = pl.pallas_call(
        body,
        out_shape=out_shape,
        grid=(10,),
        in_specs=[
            pl.BlockSpec((blk, D), lambda i: (i, 0)),
            pl.BlockSpec((blk, D), lambda i: (i, 0)),
            pl.BlockSpec((D, D), lambda i: (0, 0)),
            pl.BlockSpec((1, D), lambda i: (0, 0)),
            pl.BlockSpec((D, D), lambda i: (0, 0)),
            pl.BlockSpec((1, D), lambda i: (0, 0)),
        ],
        out_specs=out_specs,
    )(h, aggr, w1, b1.reshape(1, D), w2, b2.reshape(1, D))
    if want_relu:
        return res[0], res[1], res[2]
    return res[0], None, None


def _assemble_aggr(out_lo, out_hi):
    lo = jnp.concatenate([out_lo[0:HALF], out_lo[ACC_ROWS:ACC_ROWS + HALF]], 0)
    hi = jnp.concatenate([out_hi[0:HALF], out_hi[ACC_ROWS:ACC_ROWS + HALF]], 0)
    return jnp.concatenate([lo, hi], axis=1)


def kernel(x, edge_index, W11, b11, W12, b12, W21, b21, W22, b22, W31, b31,
           W32, b32):
    src = jnp.pad(edge_index[0], (0, E_PAD - E))
    dst = jnp.pad(edge_index[1], (0, E_PAD - E), constant_values=N)
    zeros = jnp.zeros((TPT, DH), jnp.float32)

    r_lo, r_hi = _relu_tc(x)
    aggr = _assemble_aggr(*_sc_aggregate(r_lo, r_hi, src, dst, zeros))
    h, r_lo, r_hi = _mlp_tc(x, aggr, W11, b11, W12, b12,
                            do_elu=True, want_relu=True)

    aggr = _assemble_aggr(*_sc_aggregate(r_lo, r_hi, src, dst, zeros))
    h, r_lo, r_hi = _mlp_tc(h, aggr, W21, b21, W22, b22,
                            do_elu=True, want_relu=True)

    aggr = _assemble_aggr(*_sc_aggregate(r_lo, r_hi, src, dst, zeros))
    h, _, _ = _mlp_tc(h, aggr, W31, b31, W32, b32,
                      do_elu=False, want_relu=False)
    return h
